# baseline (speedup 1.0000x reference)
import functools

import jax
import jax.numpy as jnp
from jax import lax
from jax.experimental import pallas as pl
from jax.experimental.pallas import tpu as pltpu
from jax.experimental.pallas import tpu_sc as plsc

DIM = 64
BATCH = 16384
NC = 2
NS = 16
NW = NC * NS
B_PER_W = BATCH // NW
LANES = 16


def _make_gather():
    mesh = plsc.VectorSubcoreMesh(core_axis_name="c", subcore_axis_name="s")

    @functools.partial(
        pl.kernel,
        mesh=mesh,
        out_type=jax.ShapeDtypeStruct((BATCH, DIM), jnp.float32),
        scratch_types=[
            pltpu.VMEM((B_PER_W,), jnp.int32),
            pltpu.SemaphoreType.DMA,
        ],
        compiler_params=pltpu.CompilerParams(use_tc_tiling_on_sc=True),
    )
    def gather_kernel(table_hbm, idx_hbm, out_hbm, idx_v, sem):
        wid = lax.axis_index("s") * NC + lax.axis_index("c")
        base = wid * B_PER_W
        pltpu.sync_copy(idx_hbm.at[pl.ds(base, B_PER_W)], idx_v)

        def body(k, carry):
            vec = idx_v[pl.ds(k * LANES, LANES)]
            for lane in range(LANES):
                r = vec[lane]
                b = base + k * LANES + lane
                pltpu.async_copy(
                    table_hbm.at[pl.ds(r, 1)],
                    out_hbm.at[pl.ds(b, 1)],
                    sem,
                )
            return carry

        lax.fori_loop(0, B_PER_W // LANES, body, 0)
        pltpu.make_async_copy(
            table_hbm.at[pl.ds(0, B_PER_W)],
            out_hbm.at[pl.ds(base, B_PER_W)],
            sem,
        ).wait()

    return gather_kernel


_gather = _make_gather()


def kernel(t, emb):
    return _gather(emb, t)


# speedup vs baseline: 3.6575x; 3.6575x over previous
import functools

import jax
import jax.numpy as jnp
from jax import lax
from jax.experimental import pallas as pl
from jax.experimental.pallas import tpu as pltpu
from jax.experimental.pallas import tpu_sc as plsc

DIM = 64
PADDED = 128
BATCH = 16384
NC = 2
NS = 16
NW = NC * NS
B_PER_W = BATCH // NW
CHUNK = 128
N_CHUNKS = B_PER_W // CHUNK


def _make_gather():
    mesh = plsc.VectorSubcoreMesh(core_axis_name="c", subcore_axis_name="s")

    @functools.partial(
        pl.kernel,
        mesh=mesh,
        out_type=jax.ShapeDtypeStruct((BATCH, PADDED), jnp.float32),
        scratch_types=[
            pltpu.VMEM((B_PER_W,), jnp.int32),
            pltpu.VMEM((B_PER_W, PADDED), jnp.float32),
            pltpu.SemaphoreType.DMA,
            pltpu.SemaphoreType.DMA,
        ],
        compiler_params=pltpu.CompilerParams(use_tc_tiling_on_sc=True),
    )
    def gather_kernel(table_hbm, idx_hbm, out_hbm, idx_v, rows_v, g_sem, o_sem):
        wid = lax.axis_index("s") * NC + lax.axis_index("c")
        base = wid * B_PER_W
        pltpu.sync_copy(idx_hbm.at[pl.ds(base, B_PER_W)], idx_v)
        gathers = [
            pltpu.async_copy(
                table_hbm.at[idx_v.at[pl.ds(j * CHUNK, CHUNK)]],
                rows_v.at[pl.ds(j * CHUNK, CHUNK)],
                g_sem,
            )
            for j in range(N_CHUNKS)
        ]
        outs = []
        for j in range(N_CHUNKS):
            gathers[j].wait()
            outs.append(
                pltpu.async_copy(
                    rows_v.at[pl.ds(j * CHUNK, CHUNK)],
                    out_hbm.at[pl.ds(base + j * CHUNK, CHUNK)],
                    o_sem,
                )
            )
        for o in outs:
            o.wait()

    return gather_kernel


_gather = _make_gather()


def kernel(t, emb):
    table = jnp.pad(emb, ((0, 0), (0, PADDED - DIM)))
    return _gather(table, t)[:, :DIM]


# speedup vs baseline: 3.6583x; 1.0002x over previous
import functools

import jax
import jax.numpy as jnp
from jax import lax
from jax.experimental import pallas as pl
from jax.experimental.pallas import tpu as pltpu
from jax.experimental.pallas import tpu_sc as plsc

DIM = 64
PADDED = 128
BATCH = 16384
NC = 2
NS = 16
NW = NC * NS
B_PER_W = BATCH // NW
CHUNK = 128
N_CHUNKS = B_PER_W // CHUNK


def _make_gather():
    mesh = plsc.VectorSubcoreMesh(core_axis_name="c", subcore_axis_name="s")

    @functools.partial(
        pl.kernel,
        mesh=mesh,
        out_type=jax.ShapeDtypeStruct((BATCH, PADDED), jnp.float32),
        scratch_types=[
            pltpu.VMEM((B_PER_W,), jnp.int32),
            pltpu.VMEM((B_PER_W, PADDED), jnp.float32),
            pltpu.SemaphoreType.DMA,
            pltpu.SemaphoreType.DMA,
        ],
        compiler_params=pltpu.CompilerParams(use_tc_tiling_on_sc=True),
    )
    def gather_kernel(table_hbm, idx_hbm, out_hbm, idx_v, rows_v, g_sem, o_sem):
        wid = lax.axis_index("s") * NC + lax.axis_index("c")
        base = wid * B_PER_W
        pltpu.sync_copy(idx_hbm.at[pl.ds(base, B_PER_W)], idx_v)
        gathers = [
            pltpu.async_copy(
                table_hbm.at[idx_v.at[pl.ds(j * CHUNK, CHUNK)]],
                rows_v.at[pl.ds(j * CHUNK, CHUNK)],
                g_sem,
            )
            for j in range(N_CHUNKS)
        ]
        outs = []
        for j in range(N_CHUNKS):
            gathers[j].wait()
            outs.append(
                pltpu.async_copy(
                    rows_v.at[pl.ds(j * CHUNK, CHUNK)],
                    out_hbm.at[pl.ds(base + j * CHUNK, CHUNK)],
                    o_sem,
                )
            )
        for o in outs:
            o.wait()

    return gather_kernel


_gather = _make_gather()


def kernel(t, emb):
    table = jax.lax.optimization_barrier(jnp.pad(emb, ((0, 0), (0, PADDED - DIM))))
    return _gather(table, t)[:, :DIM]
